# Initial kernel scaffold; baseline (speedup 1.0000x reference)
import jax
import jax.numpy as jnp
from jax.experimental import pallas as pl


def kernel(x, edge_index, W_self1, W_neigh1, b1, W_self2, W_neigh2, b2, W_self3, W_neigh3, b3):
    raise NotImplementedError("write your pallas kernel here")


# speedup vs baseline: 3.8353x; 3.8353x over previous
import functools

import jax
import jax.numpy as jnp
from jax import lax
from jax.experimental import pallas as pl
from jax.experimental.pallas import tpu as pltpu
from jax.experimental.pallas import tpu_sc as plsc

N = 10000
E = 320000
D = 128
H = 128
C = 64
W = 128

NC = 2
NS = 16
NW = NC * NS

N_PAD = 10240
RPT = N_PAD // NS
CHUNK = 128
N_CHUNKS = -(-E // (NW * CHUNK))
E_PAD = NW * CHUNK * N_CHUNKS
E_PER_W = E_PAD // NW

_MESH = plsc.VectorSubcoreMesh(core_axis_name="c", subcore_axis_name="s")


@functools.partial(
    pl.kernel,
    mesh=_MESH,
    out_type=jax.ShapeDtypeStruct((NC, N_PAD, W), jnp.float32),
    scratch_types=[
        pltpu.VMEM((CHUNK,), jnp.int32),
        pltpu.VMEM((CHUNK,), jnp.int32),
        pltpu.VMEM((CHUNK, W), jnp.float32),
        pltpu.VMEM_SHARED((N_PAD, W), jnp.float32),
        pltpu.SemaphoreType.DMA,
    ],
)
def _segsum(table_hbm, src_hbm, dst_hbm, zeros_hbm, out_hbm,
            sidx, didx, rows, acc, sem):
    cid = lax.axis_index("c")
    sid = lax.axis_index("s")
    wid = sid * NC + cid

    pltpu.sync_copy(zeros_hbm, acc.at[pl.ds(sid * RPT, RPT)])
    plsc.subcore_barrier()

    def body(i, carry):
        base = pl.multiple_of(wid * E_PER_W + i * CHUNK, CHUNK)
        pltpu.sync_copy(src_hbm.at[pl.ds(base, CHUNK)], sidx)
        pltpu.sync_copy(dst_hbm.at[pl.ds(base, CHUNK)], didx)
        pltpu.async_copy(table_hbm.at[sidx], rows, sem).wait()
        pltpu.sync_copy(rows, acc.at[didx], add=True)
        return carry

    lax.fori_loop(0, N_CHUNKS, body, 0)
    plsc.subcore_barrier()

    pltpu.sync_copy(acc.at[pl.ds(sid * RPT, RPT)],
                    out_hbm.at[cid, pl.ds(sid * RPT, RPT)])


@functools.partial(
    pl.kernel,
    mesh=_MESH,
    out_type=jax.ShapeDtypeStruct((NC, N_PAD, W), jnp.float32),
    scratch_types=[
        pltpu.VMEM((CHUNK,), jnp.int32),
        pltpu.VMEM((CHUNK, W), jnp.float32),
        pltpu.VMEM_SHARED((N_PAD, W), jnp.float32),
    ],
)
def _degcount(dst_hbm, ones_hbm, zeros_hbm, out_hbm, didx, ones_rows, acc):
    cid = lax.axis_index("c")
    sid = lax.axis_index("s")
    wid = sid * NC + cid

    pltpu.sync_copy(zeros_hbm, acc.at[pl.ds(sid * RPT, RPT)])
    pltpu.sync_copy(ones_hbm, ones_rows)
    plsc.subcore_barrier()

    def body(i, carry):
        base = pl.multiple_of(wid * E_PER_W + i * CHUNK, CHUNK)
        pltpu.sync_copy(dst_hbm.at[pl.ds(base, CHUNK)], didx)
        pltpu.sync_copy(ones_rows, acc.at[didx], add=True)
        return carry

    lax.fori_loop(0, N_CHUNKS, body, 0)
    plsc.subcore_barrier()

    pltpu.sync_copy(acc.at[pl.ds(sid * RPT, RPT)],
                    out_hbm.at[cid, pl.ds(sid * RPT, RPT)])


BM = 1024
_GRID = (N_PAD // BM,)


def _tc_pre1(xp, Wn, Ws, b):
    def body(x_ref, wn_ref, ws_ref, b_ref, t_ref, s_ref):
        xb = x_ref[...]
        t_ref[...] = jnp.dot(xb, wn_ref[...], preferred_element_type=jnp.float32)
        s_ref[...] = jnp.dot(xb, ws_ref[...],
                             preferred_element_type=jnp.float32) + b_ref[...]

    return pl.pallas_call(
        body,
        grid=_GRID,
        in_specs=[
            pl.BlockSpec((BM, D), lambda i: (i, 0)),
            pl.BlockSpec((D, H), lambda i: (0, 0)),
            pl.BlockSpec((D, H), lambda i: (0, 0)),
            pl.BlockSpec((1, H), lambda i: (0, 0)),
        ],
        out_specs=[
            pl.BlockSpec((BM, W), lambda i: (i, 0)),
            pl.BlockSpec((BM, H), lambda i: (i, 0)),
        ],
        out_shape=[
            jax.ShapeDtypeStruct((N_PAD, W), jnp.float32),
            jax.ShapeDtypeStruct((N_PAD, H), jnp.float32),
        ],
    )(xp, Wn, Ws, b)


def _tc_dinv(pd0, pd1):
    def body(p0_ref, p1_ref, o_ref):
        deg = p0_ref[:, :1] + p1_ref[:, :1]
        o_ref[...] = 1.0 / jnp.maximum(deg, 1.0)

    return pl.pallas_call(
        body,
        grid=_GRID,
        in_specs=[
            pl.BlockSpec((BM, W), lambda i: (i, 0)),
            pl.BlockSpec((BM, W), lambda i: (i, 0)),
        ],
        out_specs=pl.BlockSpec((BM, 1), lambda i: (i, 0)),
        out_shape=jax.ShapeDtypeStruct((N_PAD, 1), jnp.float32),
    )(pd0, pd1)


def _tc_postpre(p0, p1, s, dinv, Wn, Ws, b, cols_out):
    def body(p0_ref, p1_ref, s_ref, dinv_ref, wn_ref, ws_ref, b_ref,
             t_ref, s2_ref):
        p = p0_ref[...] + p1_ref[...]
        h = jnp.maximum(s_ref[...] + p * dinv_ref[...], 0.0)
        t_ref[...] = jnp.dot(h, wn_ref[...], preferred_element_type=jnp.float32)
        s2_ref[...] = jnp.dot(h, ws_ref[...],
                              preferred_element_type=jnp.float32) + b_ref[...]

    return pl.pallas_call(
        body,
        grid=_GRID,
        in_specs=[
            pl.BlockSpec((BM, W), lambda i: (i, 0)),
            pl.BlockSpec((BM, W), lambda i: (i, 0)),
            pl.BlockSpec((BM, H), lambda i: (i, 0)),
            pl.BlockSpec((BM, 1), lambda i: (i, 0)),
            pl.BlockSpec((H, W), lambda i: (0, 0)),
            pl.BlockSpec((H, cols_out), lambda i: (0, 0)),
            pl.BlockSpec((1, cols_out), lambda i: (0, 0)),
        ],
        out_specs=[
            pl.BlockSpec((BM, W), lambda i: (i, 0)),
            pl.BlockSpec((BM, cols_out), lambda i: (i, 0)),
        ],
        out_shape=[
            jax.ShapeDtypeStruct((N_PAD, W), jnp.float32),
            jax.ShapeDtypeStruct((N_PAD, cols_out), jnp.float32),
        ],
    )(p0, p1, s, dinv, Wn, Ws, b)


def _tc_post3(p0, p1, s, dinv):
    def body(p0_ref, p1_ref, s_ref, dinv_ref, o_ref):
        p = p0_ref[:, :C] + p1_ref[:, :C]
        o_ref[...] = s_ref[...] + p * dinv_ref[...]

    return pl.pallas_call(
        body,
        grid=_GRID,
        in_specs=[
            pl.BlockSpec((BM, W), lambda i: (i, 0)),
            pl.BlockSpec((BM, W), lambda i: (i, 0)),
            pl.BlockSpec((BM, C), lambda i: (i, 0)),
            pl.BlockSpec((BM, 1), lambda i: (i, 0)),
        ],
        out_specs=pl.BlockSpec((BM, C), lambda i: (i, 0)),
        out_shape=jax.ShapeDtypeStruct((N_PAD, C), jnp.float32),
    )(p0, p1, s, dinv)


def kernel(x, edge_index, W_self1, W_neigh1, b1, W_self2, W_neigh2, b2,
           W_self3, W_neigh3, b3):
    src = edge_index[0].astype(jnp.int32)
    dst = edge_index[1].astype(jnp.int32)
    src_p = jnp.concatenate([src, jnp.zeros((E_PAD - E,), jnp.int32)])
    dst_p = jnp.concatenate([dst, jnp.full((E_PAD - E,), N_PAD - 1, jnp.int32)])

    xp = jnp.zeros((N_PAD, D), jnp.float32).at[:N].set(x)
    zeros = jnp.zeros((RPT, W), jnp.float32)
    ones = jnp.ones((CHUNK, W), jnp.float32)

    b1r = b1.reshape(1, H)
    b2r = b2.reshape(1, H)
    b3r = b3.reshape(1, C)
    Wn3p = jnp.zeros((H, W), jnp.float32).at[:, :C].set(W_neigh3)

    pd = _degcount(dst_p, ones, zeros)
    dinv = _tc_dinv(pd[0], pd[1])

    t1, s1 = _tc_pre1(xp, W_neigh1, W_self1, b1r)
    p1 = _segsum(t1, src_p, dst_p, zeros)
    t2, s2 = _tc_postpre(p1[0], p1[1], s1, dinv, W_neigh2, W_self2, b2r, H)
    p2 = _segsum(t2, src_p, dst_p, zeros)
    t3, s3 = _tc_postpre(p2[0], p2[1], s2, dinv, Wn3p, W_self3, b3r, C)
    p3 = _segsum(t3, src_p, dst_p, zeros)
    out = _tc_post3(p3[0], p3[1], s3, dinv)
    return out[:N]
